# baseline (speedup 1.0000x reference)
import jax
import jax.numpy as jnp
from jax import lax
from jax.experimental import pallas as pl
from jax.experimental.pallas import tpu as pltpu
from jax.experimental.pallas import tpu_sc as plsc


def _copy_body(table_hbm, out_hbm):
    wid = lax.axis_index("s") * 2 + lax.axis_index("c")

    @pl.when(wid == 0)
    def _():
        pltpu.sync_copy(table_hbm, out_hbm)


def kernel(x, a, onehot_encoding, parameters_encoding_matrix):
    mesh = plsc.VectorSubcoreMesh(core_axis_name="c", subcore_axis_name="s")
    run = pl.kernel(
        _copy_body,
        out_type=jax.ShapeDtypeStruct(onehot_encoding.shape, onehot_encoding.dtype),
        mesh=mesh,
    )
    return run(onehot_encoding)


# speedup vs baseline: 1.1626x; 1.1626x over previous
import jax
import jax.numpy as jnp
from jax import lax
from jax.experimental import pallas as pl
from jax.experimental.pallas import tpu as pltpu
from jax.experimental.pallas import tpu_sc as plsc


def _copy_body(table_hbm, out_hbm):
    pltpu.sync_copy(table_hbm, out_hbm)


def kernel(x, a, onehot_encoding, parameters_encoding_matrix):
    mesh = plsc.ScalarSubcoreMesh(axis_name="c", num_cores=1)
    run = pl.kernel(
        _copy_body,
        out_type=jax.ShapeDtypeStruct(onehot_encoding.shape, onehot_encoding.dtype),
        mesh=mesh,
    )
    return run(onehot_encoding)
